# baseline (speedup 1.0000x reference)
import functools

import jax
import jax.numpy as jnp
from jax import lax
from jax.experimental import pallas as pl
from jax.experimental.pallas import tpu as pltpu


def _pick_tile(n, target):
    t = min(target, max(8, -(-n // 8) * 8))
    t = (t // 8) * 8
    while t >= 8:
        if n % t == 0:
            return t
        t -= 8
    return target


def _fused_kernel(x_ref, nb_ref, e_ref, ei_ref, u_ref, w_ref, b_ref,
                  out_ref,
                  npart_ref, epart_ref, cnt_ref, lo_ref, hi_ref, iota_n_ref,
                  *, n_node_chunks, n_edge_chunks,
                  n_valid, e_valid, mask_n, mask_e):
    i = pl.program_id(0)
    n_total = n_node_chunks + n_edge_chunks

    B = npart_ref.shape[0]
    TN = x_ref.shape[0]
    TE = e_ref.shape[0]

    @pl.when(i == 0)
    def _init():
        npart_ref[...] = jnp.zeros_like(npart_ref)
        epart_ref[...] = jnp.zeros_like(epart_ref)
        cnt_ref[...] = jnp.zeros_like(cnt_ref)
        iota_n_ref[...] = lax.broadcasted_iota(jnp.int32, (B, TN), 0)

    dims = (((1,), (0,)), ((), ()))

    @pl.when(i < n_node_chunks)
    def _node():
        oh = nb_ref[...] == iota_n_ref[...]
        if mask_n:
            pos = i * TN + lax.broadcasted_iota(jnp.int32, (1, TN), 1)
            oh = jnp.logical_and(oh, pos < n_valid)
        ohf = oh.astype(jnp.float32)
        cnt_ref[...] += jnp.sum(ohf, axis=1, keepdims=True)
        npart_ref[...] += lax.dot_general(
            ohf.astype(jnp.bfloat16), x_ref[...].astype(jnp.bfloat16),
            dims, preferred_element_type=jnp.float32)

    @pl.when(i == n_node_chunks - 1)
    def _bounds():
        r = lax.broadcasted_iota(jnp.int32, (B, B), 0)
        c = lax.broadcasted_iota(jnp.int32, (B, B), 1)
        tri = (c < r).astype(jnp.float32)
        lo = jnp.dot(tri, cnt_ref[...], preferred_element_type=jnp.float32,
                     precision=lax.Precision.HIGHEST)
        lo_ref[...] = lo.astype(jnp.int32)
        hi_ref[...] = (lo + cnt_ref[...]).astype(jnp.int32)

    @pl.when(i >= n_node_chunks)
    def _edge():
        tgt = ei_ref[...]
        oh = jnp.logical_and(tgt >= lo_ref[...], tgt < hi_ref[...])
        if mask_e:
            j = i - n_node_chunks
            pos = j * TE + lax.broadcasted_iota(jnp.int32, (1, TE), 1)
            oh = jnp.logical_and(oh, pos < e_valid)
        epart_ref[...] += lax.dot_general(
            oh.astype(jnp.bfloat16), e_ref[...].astype(jnp.bfloat16),
            dims, preferred_element_type=jnp.float32)

    @pl.when(i == n_total - 1)
    def _epilogue():
        feats = jnp.concatenate(
            [npart_ref[...], epart_ref[...], u_ref[...]], axis=1)
        y = jnp.dot(feats, w_ref[...], preferred_element_type=jnp.float32)
        out_ref[...] = jnp.maximum(y + b_ref[...], 0.0)


def kernel(x, edge_index, edge_attr, u, batch, W, b):
    N, Fx = x.shape
    E, Fe = edge_attr.shape
    B, Fu = u.shape
    out_dim = W.shape[1]

    tile_n = _pick_tile(N, 8192)
    tile_e = _pick_tile(E, 16384)

    batch = batch.astype(jnp.int32)
    nb = batch.reshape(1, N)
    ei1 = edge_index[1].astype(jnp.int32).reshape(1, E)

    n_node_chunks = pl.cdiv(N, tile_n)
    n_edge_chunks = pl.cdiv(E, tile_e)
    n_total = n_node_chunks + n_edge_chunks

    x_map = lambda i: (jnp.minimum(i, n_node_chunks - 1), 0)
    nb_map = lambda i: (0, jnp.minimum(i, n_node_chunks - 1))
    e_map = lambda i: (jnp.clip(i - n_node_chunks, 0, n_edge_chunks - 1), 0)
    ei_map = lambda i: (0, jnp.clip(i - n_node_chunks, 0, n_edge_chunks - 1))
    const_map = lambda i: (0, 0)

    body = functools.partial(
        _fused_kernel,
        n_node_chunks=n_node_chunks, n_edge_chunks=n_edge_chunks,
        n_valid=N, e_valid=E,
        mask_n=(N % tile_n != 0), mask_e=(E % tile_e != 0))

    vmem_need = (2 * 4 * (tile_n * (Fx + 1) + tile_e * (Fe + 1))
                 + 2 * (tile_n * Fx + tile_e * Fe)
                 + 7 * B * tile_n + 3 * B * tile_e
                 + 4 * B * (Fx + Fe + Fu + out_dim)
                 + 4 * (Fx + Fe + Fu) * out_dim + (6 << 20))
    vmem_limit = int(min(max(vmem_need, 32 << 20), 100 << 20))

    cost = pl.CostEstimate(
        flops=2 * B * (N * Fx + E * Fe) + 2 * B * (Fx + Fe + Fu) * out_dim,
        transcendentals=0,
        bytes_accessed=4 * ((Fx + 1) * N + (Fe + 1) * E + B * Fu
                            + (Fx + Fe + Fu + 1) * out_dim + B * out_dim),
    )

    return pl.pallas_call(
        body,
        out_shape=jax.ShapeDtypeStruct((B, out_dim), jnp.float32),
        grid_spec=pltpu.PrefetchScalarGridSpec(
            num_scalar_prefetch=0,
            grid=(n_total,),
            in_specs=[
                pl.BlockSpec((tile_n, Fx), x_map),
                pl.BlockSpec((1, tile_n), nb_map),
                pl.BlockSpec((tile_e, Fe), e_map),
                pl.BlockSpec((1, tile_e), ei_map),
                pl.BlockSpec((B, Fu), const_map),
                pl.BlockSpec((Fx + Fe + Fu, out_dim), const_map),
                pl.BlockSpec((1, out_dim), const_map),
            ],
            out_specs=pl.BlockSpec((B, out_dim), const_map),
            scratch_shapes=[
                pltpu.VMEM((B, Fx), jnp.float32),
                pltpu.VMEM((B, Fe), jnp.float32),
                pltpu.VMEM((B, 1), jnp.float32),
                pltpu.VMEM((B, 1), jnp.int32),
                pltpu.VMEM((B, 1), jnp.int32),
                pltpu.VMEM((B, tile_n), jnp.int32),
            ],
        ),
        compiler_params=pltpu.CompilerParams(
            dimension_semantics=("arbitrary",),
            vmem_limit_bytes=vmem_limit),
        cost_estimate=cost,
    )(x, nb, edge_attr, ei1, u, W, b.reshape(1, out_dim))
